# Initial kernel scaffold; baseline (speedup 1.0000x reference)
import jax
import jax.numpy as jnp
from jax.experimental import pallas as pl


def kernel(x, edge_index, edge_attr, batch, global_feat, emb_n_W, emb_n_b, emb_e_W, emb_e_b, conv_W, conv_att, conv_bias, bn_a_gamma, bn_a_beta, bn_n_gamma, bn_n_beta, comp_node_W, comp_node_b, comp_att_W, comp_att_b):
    raise NotImplementedError("write your pallas kernel here")


# speedup vs baseline: 3.7187x; 3.7187x over previous
import jax
import jax.numpy as jnp
from jax.experimental import pallas as pl

EPS = 1e-5


def _sp(x):
    return jax.nn.softplus(x)


def _affine_kernel(x_ref, w_ref, b_ref, o_ref):
    o_ref[:] = jnp.dot(x_ref[:], w_ref[:], preferred_element_type=jnp.float32) + b_ref[:]


def _gather_kernel(idx_ref, tab_ref, o_ref):
    idx = idx_ref[:]
    nrows = tab_ref.shape[0]
    oh = (idx == jax.lax.broadcasted_iota(jnp.int32, (idx.shape[0], nrows), 1)).astype(jnp.float32)
    o_ref[:] = jnp.dot(oh, tab_ref[:], preferred_element_type=jnp.float32)


def _edge_kernel(hs_ref, hd_ref, ea_ref, w_ref, atti_ref, attj_ref, hj_ref, ar_ref):
    heads = ar_ref.shape[1]
    nh = hs_ref.shape[1]
    xi = jnp.concatenate([hd_ref[:], ea_ref[:]], axis=1)
    xj = jnp.concatenate([hs_ref[:], ea_ref[:]], axis=1)
    w = w_ref[:]
    hi = _sp(jnp.dot(xi, w, preferred_element_type=jnp.float32))
    hj = _sp(jnp.dot(xj, w, preferred_element_type=jnp.float32))
    hj_ref[:] = hj
    t = hi * atti_ref[:] + hj * attj_ref[:]
    d = jax.lax.broadcasted_iota(jnp.int32, (heads * nh, heads), 0) // nh
    hh = jax.lax.broadcasted_iota(jnp.int32, (heads * nh, heads), 1)
    sel = (d == hh).astype(jnp.float32)
    ar_ref[:] = _sp(jnp.dot(t, sel, preferred_element_type=jnp.float32))


def _bnstat1_kernel(a_ref, o_ref):
    @pl.when(pl.program_id(0) == 0)
    def _zero():
        o_ref[:] = jnp.zeros_like(o_ref)

    o_ref[:] += jnp.sum(a_ref[:], axis=0, keepdims=True)


def _bnstat2_kernel(a_ref, s1_ref, o_ref):
    @pl.when(pl.program_id(0) == 0)
    def _zero():
        o_ref[:] = jnp.zeros_like(o_ref)

    etot = jnp.float32(pl.num_programs(0) * a_ref.shape[0])
    mu = s1_ref[:] / etot
    d = a_ref[:] - mu
    o_ref[:] += jnp.sum(d * d, axis=0, keepdims=True)


def _scatter_kernel(idx_ref, hj_ref, ar_ref, st_ref, g_ref, b_ref, o_ref):
    @pl.when(pl.program_id(0) == 0)
    def _zero():
        o_ref[:] = jnp.zeros_like(o_ref)

    nnodes = o_ref.shape[0]
    eblk = hj_ref.shape[0]
    heads = ar_ref.shape[1]
    nh = hj_ref.shape[1] // heads
    idxr = idx_ref[pl.ds(pl.program_id(0), 1), :]
    oh = (jax.lax.broadcasted_iota(jnp.int32, (nnodes, eblk), 0) == idxr).astype(jnp.float32)
    etot = jnp.float32(pl.num_programs(0) * eblk)
    st = st_ref[:]
    mu = st[0:1, :] / etot
    var = st[1:2, :] / etot
    an = (ar_ref[:] - mu) / jnp.sqrt(var + EPS) * g_ref[:] + b_ref[:]
    e = jnp.exp(_sp(an))
    d = jax.lax.broadcasted_iota(jnp.int32, (heads, heads * nh), 1) // nh
    hh = jax.lax.broadcasted_iota(jnp.int32, (heads, heads * nh), 0)
    selT = (d == hh).astype(jnp.float32)
    erep = jnp.dot(e, selT, preferred_element_type=jnp.float32)
    vals = jnp.concatenate([hj_ref[:] * erep, e], axis=1)
    o_ref[:] += jnp.dot(oh, vals, preferred_element_type=jnp.float32)


def _node_kernel(ns_ref, bias_ref, g_ref, b_ref, o_ref):
    nh = o_ref.shape[1]
    heads = ns_ref.shape[1] // (nh + 1)
    ns = ns_ref[:]
    acc = jnp.zeros((ns.shape[0], nh), jnp.float32)
    for h in range(heads):
        m = ns[:, h * nh:(h + 1) * nh]
        s = ns[:, heads * nh + h:heads * nh + h + 1]
        acc = acc + m / (s + 1e-16)
    hc = acc / heads + bias_ref[:]
    mu = jnp.mean(hc, axis=0, keepdims=True)
    var = jnp.mean((hc - mu) * (hc - mu), axis=0, keepdims=True)
    o_ref[:] = _sp((hc - mu) / jnp.sqrt(var + EPS) * g_ref[:] + b_ref[:])


def _comp_kernel(h_ref, gf_ref, w1_ref, b1_ref, w2_ref, b2_ref, e_ref):
    z = jnp.concatenate([h_ref[:], gf_ref[:]], axis=1)
    a1 = _sp(jnp.dot(z, w1_ref[:], preferred_element_type=jnp.float32) + b1_ref[:])
    a = jnp.dot(a1, w2_ref[:], preferred_element_type=jnp.float32) + b2_ref[:]
    amax = jnp.max(a)
    e_ref[:] = jnp.exp(a - amax)


def _pool_kernel(brow_ref, bcol_ref, h_ref, e_ref, o_ref):
    ng = o_ref.shape[0]
    n = h_ref.shape[0]
    brow = brow_ref[:]
    oh = (jax.lax.broadcasted_iota(jnp.int32, (ng, n), 0) == brow).astype(jnp.float32)
    e = e_ref[:]
    s = jnp.dot(oh, e, preferred_element_type=jnp.float32)
    bcol = bcol_ref[:]
    oh2 = (bcol == jax.lax.broadcasted_iota(jnp.int32, (n, ng), 1)).astype(jnp.float32)
    sn = jnp.dot(oh2, s, preferred_element_type=jnp.float32)
    w = e / (sn + 1e-16)
    o_ref[:] = jnp.dot(oh, h_ref[:] * w, preferred_element_type=jnp.float32)


def _full_spec(shape):
    return pl.BlockSpec(shape, lambda *a: tuple(0 for _ in shape))


def _affine(x, w, b, eblk=None):
    rows, kin = x.shape
    kout = w.shape[1]
    b2 = b.reshape(1, kout)
    if eblk is None or rows % eblk != 0:
        eblk = rows
    grid = (rows // eblk,)
    return pl.pallas_call(
        _affine_kernel,
        grid=grid,
        in_specs=[
            pl.BlockSpec((eblk, kin), lambda i: (i, 0)),
            _full_spec((kin, kout)),
            _full_spec((1, kout)),
        ],
        out_specs=pl.BlockSpec((eblk, kout), lambda i: (i, 0)),
        out_shape=jax.ShapeDtypeStruct((rows, kout), jnp.float32),
    )(x, w, b2)


def _gather(tab, idx_col, eblk=None):
    rows = idx_col.shape[0]
    nrows, dim = tab.shape
    if eblk is None or rows % eblk != 0:
        eblk = rows
    grid = (rows // eblk,)
    return pl.pallas_call(
        _gather_kernel,
        grid=grid,
        in_specs=[
            pl.BlockSpec((eblk, 1), lambda i: (i, 0)),
            _full_spec((nrows, dim)),
        ],
        out_specs=pl.BlockSpec((eblk, dim), lambda i: (i, 0)),
        out_shape=jax.ShapeDtypeStruct((rows, dim), jnp.float32),
    )(idx_col, tab)


def kernel(x, edge_index, edge_attr, batch, global_feat, emb_n_W, emb_n_b,
           emb_e_W, emb_e_b, conv_W, conv_att, conv_bias, bn_a_gamma,
           bn_a_beta, bn_n_gamma, bn_n_beta, comp_node_W, comp_node_b,
           comp_att_W, comp_att_b):
    n = x.shape[0]
    e = edge_attr.shape[0]
    ng = global_feat.shape[0]
    nh = emb_n_W.shape[1]
    heads = conv_att.shape[1]
    nl = conv_W.shape[0]
    eblk = 256 if e % 256 == 0 else None

    src = edge_index[0]
    dst = edge_index[1]
    src_col = src.reshape(e, 1)
    dst_col = dst.reshape(e, 1)
    blk = eblk if eblk is not None else e
    dst2d = dst.reshape(e // blk, blk)

    h = _affine(x, emb_n_W, emb_n_b)
    ea = _affine(edge_attr, emb_e_W, emb_e_b, eblk=eblk)

    for l in range(nl):
        hs = _gather(h, src_col, eblk=eblk)
        hd = _gather(h, dst_col, eblk=eblk)
        atti = conv_att[l][:, :nh].reshape(1, heads * nh)
        attj = conv_att[l][:, nh:].reshape(1, heads * nh)
        hj, araw = pl.pallas_call(
            _edge_kernel,
            grid=(e // blk,),
            in_specs=[
                pl.BlockSpec((blk, nh), lambda i: (i, 0)),
                pl.BlockSpec((blk, nh), lambda i: (i, 0)),
                pl.BlockSpec((blk, nh), lambda i: (i, 0)),
                _full_spec((2 * nh, heads * nh)),
                _full_spec((1, heads * nh)),
                _full_spec((1, heads * nh)),
            ],
            out_specs=[
                pl.BlockSpec((blk, heads * nh), lambda i: (i, 0)),
                pl.BlockSpec((blk, heads), lambda i: (i, 0)),
            ],
            out_shape=[
                jax.ShapeDtypeStruct((e, heads * nh), jnp.float32),
                jax.ShapeDtypeStruct((e, heads), jnp.float32),
            ],
        )(hs, hd, ea, conv_W[l], atti, attj)

        s1 = pl.pallas_call(
            _bnstat1_kernel,
            grid=(e // blk,),
            in_specs=[pl.BlockSpec((blk, heads), lambda i: (i, 0))],
            out_specs=_full_spec((1, heads)),
            out_shape=jax.ShapeDtypeStruct((1, heads), jnp.float32),
        )(araw)
        ssq = pl.pallas_call(
            _bnstat2_kernel,
            grid=(e // blk,),
            in_specs=[
                pl.BlockSpec((blk, heads), lambda i: (i, 0)),
                _full_spec((1, heads)),
            ],
            out_specs=_full_spec((1, heads)),
            out_shape=jax.ShapeDtypeStruct((1, heads), jnp.float32),
        )(araw, s1)
        stats = jnp.concatenate([s1, ssq], axis=0)

        nsum = pl.pallas_call(
            _scatter_kernel,
            grid=(e // blk,),
            in_specs=[
                _full_spec((e // blk, blk)),
                pl.BlockSpec((blk, heads * nh), lambda i: (i, 0)),
                pl.BlockSpec((blk, heads), lambda i: (i, 0)),
                _full_spec((2, heads)),
                _full_spec((1, heads)),
                _full_spec((1, heads)),
            ],
            out_specs=_full_spec((n, heads * nh + heads)),
            out_shape=jax.ShapeDtypeStruct((n, heads * nh + heads), jnp.float32),
        )(dst2d, hj, araw, stats, bn_a_gamma[l].reshape(1, heads),
          bn_a_beta[l].reshape(1, heads))

        h = pl.pallas_call(
            _node_kernel,
            in_specs=[
                _full_spec((n, heads * nh + heads)),
                _full_spec((1, nh)),
                _full_spec((1, nh)),
                _full_spec((1, nh)),
            ],
            out_specs=_full_spec((n, nh)),
            out_shape=jax.ShapeDtypeStruct((n, nh), jnp.float32),
        )(nsum, conv_bias[l].reshape(1, nh), bn_n_gamma[l].reshape(1, nh),
          bn_n_beta[l].reshape(1, nh))

    gf = _gather(global_feat, batch.reshape(n, 1))
    gdim = global_feat.shape[1]
    adim = comp_node_W.shape[1]
    e_pool = pl.pallas_call(
        _comp_kernel,
        in_specs=[
            _full_spec((n, nh)),
            _full_spec((n, gdim)),
            _full_spec((nh + gdim, adim)),
            _full_spec((1, adim)),
            _full_spec((adim, 1)),
            _full_spec((1, 1)),
        ],
        out_specs=_full_spec((n, 1)),
        out_shape=jax.ShapeDtypeStruct((n, 1), jnp.float32),
    )(h, gf, comp_node_W, comp_node_b.reshape(1, adim),
      comp_att_W, comp_att_b.reshape(1, 1))

    out = pl.pallas_call(
        _pool_kernel,
        in_specs=[
            _full_spec((1, n)),
            _full_spec((n, 1)),
            _full_spec((n, nh)),
            _full_spec((n, 1)),
        ],
        out_specs=_full_spec((ng, nh)),
        out_shape=jax.ShapeDtypeStruct((ng, nh), jnp.float32),
    )(batch.reshape(1, n), batch.reshape(n, 1), h, e_pool)
    return out
